# baseline (speedup 1.0000x reference)
import functools

import jax
import jax.numpy as jnp
from jax import lax
from jax.experimental import pallas as pl
from jax.experimental.pallas import tpu as pltpu
from jax.experimental.pallas import tpu_sc as plsc

NQ = 16384
NK = 4096
DF = 256
DS = 128
KNN = 3

NCHUNKS = 4
CQ = NQ // NCHUNKS

BM_A = 256
BM_C = 2048

NW = 32
CH = 32


def _topk_body(q_ref, pos_ref, idx_ref, wnb_ref):
    q = q_ref[...]
    pos = pos_ref[...]
    qq = jnp.sum(q * q, axis=1, keepdims=True)
    kk = jnp.sum(pos * pos, axis=1)[None, :]
    qk = jax.lax.dot_general(q, pos, (((1,), (1,)), ((), ())),
                             preferred_element_type=jnp.float32)
    d2 = jnp.maximum(qq + kk - 2.0 * qk, 0.0)
    bits = jax.lax.bitcast_convert_type(d2, jnp.int32)
    iota = jax.lax.broadcasted_iota(jnp.int32, d2.shape, 1)
    imax = jnp.int32(0x7FFFFFFF)
    ms, is_ = [], []
    for r in range(KNN):
        m = jnp.min(bits, axis=1)
        i = jnp.min(jnp.where(bits == m[:, None], iota, imax), axis=1)
        ms.append(m)
        is_.append(i)
        if r < KNN - 1:
            bits = jnp.where(iota == i[:, None], imax, bits)
    idx_ref[...] = jnp.stack(is_, axis=1)
    d2k = jax.lax.bitcast_convert_type(jnp.stack(ms, axis=1), jnp.float32)
    w = 1.0 / jnp.maximum(d2k, 1e-16)
    wn = w / jnp.sum(w, axis=1, keepdims=True)
    wnb = jnp.concatenate(
        [jnp.broadcast_to(wn[:, j:j + 1], (BM_A, 16)) for j in range(KNN)],
        axis=1)
    wnb_ref[...] = wnb


def _topk(pos_skip_c, pos):
    nq = pos_skip_c.shape[0]
    return pl.pallas_call(
        _topk_body,
        grid=(nq // BM_A,),
        in_specs=[
            pl.BlockSpec((BM_A, 3), lambda i: (i, 0)),
            pl.BlockSpec((NK, 3), lambda i: (0, 0)),
        ],
        out_specs=[
            pl.BlockSpec((BM_A, KNN), lambda i: (i, 0)),
            pl.BlockSpec((BM_A, KNN * 16), lambda i: (i, 0)),
        ],
        out_shape=[
            jax.ShapeDtypeStruct((nq, KNN), jnp.int32),
            jax.ShapeDtypeStruct((nq, KNN * 16), jnp.float32),
        ],
    )(pos_skip_c, pos)


def _sc_body(nq, x_hbm, idx_hbm, wnb_hbm, out_hbm, idx_v, wn_v, rows_v,
             out_v, sem):
    qpw = nq // NW
    nchunk = qpw // CH
    wid = lax.axis_index("s") * 2 + lax.axis_index("c")
    qbase = wid * qpw

    def chunk(ci, carry):
        q0 = qbase + ci * CH
        b0 = pl.multiple_of(q0 * KNN, 8)
        pltpu.sync_copy(idx_hbm.at[pl.ds(b0, CH * KNN)], idx_v)
        pltpu.sync_copy(wnb_hbm.at[pl.ds(b0, CH * KNN)], wn_v)
        pltpu.async_copy(x_hbm.at[idx_v], rows_v, sem).wait()

        def q_body(qi, carry2):
            w0 = wn_v[3 * qi, :]
            w1 = wn_v[3 * qi + 1, :]
            w2 = wn_v[3 * qi + 2, :]
            for dv in range(DF // 16):
                sl = pl.ds(dv * 16, 16)
                acc = (w0 * rows_v[3 * qi, sl]
                       + w1 * rows_v[3 * qi + 1, sl]
                       + w2 * rows_v[3 * qi + 2, sl])
                out_v[qi, sl] = acc
            return carry2

        lax.fori_loop(0, CH, q_body, 0, unroll=False)
        pltpu.sync_copy(out_v, out_hbm.at[pl.ds(q0, CH)])
        return carry

    lax.fori_loop(0, nchunk, chunk, 0, unroll=False)


def _sc_interpolate(x, idx_flat, wnb_flat):
    nq = idx_flat.shape[0] // KNN
    mesh = plsc.VectorSubcoreMesh(core_axis_name="c", subcore_axis_name="s")
    f = functools.partial(
        pl.kernel,
        mesh=mesh,
        out_type=jax.ShapeDtypeStruct((nq, DF), jnp.float32),
        scratch_types=[
            pltpu.VMEM((CH * KNN,), jnp.int32),
            pltpu.VMEM((CH * KNN, 16), jnp.float32),
            pltpu.VMEM((CH * KNN, DF), jnp.float32),
            pltpu.VMEM((CH, DF), jnp.float32),
            pltpu.SemaphoreType.DMA,
        ],
    )(functools.partial(_sc_body, nq))
    return f(x, idx_flat, wnb_flat)


def _mlp_body(xi_ref, xs_ref, w_ref, b_ref, out_ref):
    w1 = w_ref[:DF, :]
    w2 = w_ref[DF:, :]
    h = (jax.lax.dot_general(xi_ref[...], w1, (((1,), (0,)), ((), ())),
                             preferred_element_type=jnp.float32)
         + jax.lax.dot_general(xs_ref[...], w2, (((1,), (0,)), ((), ())),
                               preferred_element_type=jnp.float32)
         + b_ref[...])
    out_ref[...] = jnp.maximum(h, 0.0)


def _mlp(xi, x_skip_c, W, b2d):
    nq = xi.shape[0]
    bm = min(BM_C, nq)
    return pl.pallas_call(
        _mlp_body,
        grid=(nq // bm,),
        in_specs=[
            pl.BlockSpec((bm, DF), lambda i: (i, 0)),
            pl.BlockSpec((bm, DS), lambda i: (i, 0)),
            pl.BlockSpec((DF + DS, DF), lambda i: (0, 0)),
            pl.BlockSpec((1, DF), lambda i: (0, 0)),
        ],
        out_specs=pl.BlockSpec((bm, DF), lambda i: (i, 0)),
        out_shape=jax.ShapeDtypeStruct((nq, DF), jnp.float32),
    )(xi, x_skip_c, W, b2d)


def kernel(x, pos, x_skip, pos_skip, W, b):
    b2d = b.reshape(1, DF)
    outs = []
    for c in range(NCHUNKS):
        sl = slice(c * CQ, (c + 1) * CQ)
        idx3, wnb = _topk(pos_skip[sl], pos)
        idx_flat = idx3.reshape(CQ * KNN)
        wnb_flat = wnb.reshape(CQ * KNN, 16)
        xi = _sc_interpolate(x, idx_flat, wnb_flat)
        outs.append(_mlp(xi, x_skip[sl], W, b2d))
    out = jnp.concatenate(outs, axis=0)
    return (out, pos_skip)


# speedup vs baseline: 1.4895x; 1.4895x over previous
import functools

import jax
import jax.numpy as jnp
from jax import lax
from jax.experimental import pallas as pl
from jax.experimental.pallas import tpu as pltpu
from jax.experimental.pallas import tpu_sc as plsc

NQ = 16384
NK = 4096
DF = 256
DS = 128
KNN = 3

NCHUNKS = 4
CQ = NQ // NCHUNKS

BM_A = 512
BM_C = 2048

NW = 32
CH = 32


def _topk_body(q_ref, pos_ref, idx_ref, wnb_ref):
    q = q_ref[...]
    pos = pos_ref[...]
    qq = jnp.sum(q * q, axis=1, keepdims=True)
    kk = jnp.sum(pos * pos, axis=1)[None, :]
    qk = jax.lax.dot_general(q, pos, (((1,), (1,)), ((), ())),
                             preferred_element_type=jnp.float32)
    d2 = jnp.maximum(qq + kk - 2.0 * qk, 0.0)
    iota_i = jax.lax.broadcasted_iota(jnp.int32, d2.shape, 1)
    iota = iota_i.astype(jnp.float32)
    inf = jnp.float32(jnp.inf)
    ms, is_ = [], []
    for r in range(KNN):
        m = jnp.min(d2, axis=1)
        i = jnp.min(jnp.where(d2 == m[:, None], iota, inf), axis=1)
        ms.append(m)
        is_.append(i)
        if r < KNN - 1:
            d2 = jnp.where(iota == i[:, None], inf, d2)
    idx_ref[...] = jnp.stack(is_, axis=1).astype(jnp.int32)
    d2k = jnp.stack(ms, axis=1)
    w = 1.0 / jnp.maximum(d2k, 1e-16)
    wn = w / jnp.sum(w, axis=1, keepdims=True)
    wnb = jnp.concatenate(
        [jnp.broadcast_to(wn[:, j:j + 1], (BM_A, 16)) for j in range(KNN)],
        axis=1)
    wnb_ref[...] = wnb


def _topk(pos_skip_c, pos):
    nq = pos_skip_c.shape[0]
    return pl.pallas_call(
        _topk_body,
        grid=(nq // BM_A,),
        in_specs=[
            pl.BlockSpec((BM_A, 3), lambda i: (i, 0)),
            pl.BlockSpec((NK, 3), lambda i: (0, 0)),
        ],
        out_specs=[
            pl.BlockSpec((BM_A, KNN), lambda i: (i, 0)),
            pl.BlockSpec((BM_A, KNN * 16), lambda i: (i, 0)),
        ],
        out_shape=[
            jax.ShapeDtypeStruct((nq, KNN), jnp.int32),
            jax.ShapeDtypeStruct((nq, KNN * 16), jnp.float32),
        ],
    )(pos_skip_c, pos)


def _sc_body(nq, x_hbm, idx_hbm, wnb_hbm, out_hbm, idx_v, wn_v, rows_v,
             out_v, sem):
    qpw = nq // NW
    nchunk = qpw // CH
    wid = lax.axis_index("s") * 2 + lax.axis_index("c")
    qbase = wid * qpw

    def chunk(ci, carry):
        q0 = qbase + ci * CH
        b0 = pl.multiple_of(q0 * KNN, 8)
        pltpu.sync_copy(idx_hbm.at[pl.ds(b0, CH * KNN)], idx_v)
        pltpu.sync_copy(wnb_hbm.at[pl.ds(b0, CH * KNN)], wn_v)
        pltpu.async_copy(x_hbm.at[idx_v], rows_v, sem).wait()

        def q_body(qi, carry2):
            w0 = wn_v[3 * qi, :]
            w1 = wn_v[3 * qi + 1, :]
            w2 = wn_v[3 * qi + 2, :]
            for dv in range(DF // 16):
                sl = pl.ds(dv * 16, 16)
                acc = (w0 * rows_v[3 * qi, sl]
                       + w1 * rows_v[3 * qi + 1, sl]
                       + w2 * rows_v[3 * qi + 2, sl])
                out_v[qi, sl] = acc
            return carry2

        lax.fori_loop(0, CH, q_body, 0, unroll=False)
        pltpu.sync_copy(out_v, out_hbm.at[pl.ds(q0, CH)])
        return carry

    lax.fori_loop(0, nchunk, chunk, 0, unroll=False)


def _sc_interpolate(x, idx_flat, wnb_flat):
    nq = idx_flat.shape[0] // KNN
    mesh = plsc.VectorSubcoreMesh(core_axis_name="c", subcore_axis_name="s")
    f = functools.partial(
        pl.kernel,
        mesh=mesh,
        out_type=jax.ShapeDtypeStruct((nq, DF), jnp.float32),
        scratch_types=[
            pltpu.VMEM((CH * KNN,), jnp.int32),
            pltpu.VMEM((CH * KNN, 16), jnp.float32),
            pltpu.VMEM((CH * KNN, DF), jnp.float32),
            pltpu.VMEM((CH, DF), jnp.float32),
            pltpu.SemaphoreType.DMA,
        ],
    )(functools.partial(_sc_body, nq))
    return f(x, idx_flat, wnb_flat)


def _mlp_body(xi_ref, xs_ref, w_ref, b_ref, out_ref):
    w1 = w_ref[:DF, :]
    w2 = w_ref[DF:, :]
    h = (jax.lax.dot_general(xi_ref[...], w1, (((1,), (0,)), ((), ())),
                             preferred_element_type=jnp.float32)
         + jax.lax.dot_general(xs_ref[...], w2, (((1,), (0,)), ((), ())),
                               preferred_element_type=jnp.float32)
         + b_ref[...])
    out_ref[...] = jnp.maximum(h, 0.0)


def _mlp(xi, x_skip_c, W, b2d):
    nq = xi.shape[0]
    bm = min(BM_C, nq)
    return pl.pallas_call(
        _mlp_body,
        grid=(nq // bm,),
        in_specs=[
            pl.BlockSpec((bm, DF), lambda i: (i, 0)),
            pl.BlockSpec((bm, DS), lambda i: (i, 0)),
            pl.BlockSpec((DF + DS, DF), lambda i: (0, 0)),
            pl.BlockSpec((1, DF), lambda i: (0, 0)),
        ],
        out_specs=pl.BlockSpec((bm, DF), lambda i: (i, 0)),
        out_shape=jax.ShapeDtypeStruct((nq, DF), jnp.float32),
    )(xi, x_skip_c, W, b2d)


def kernel(x, pos, x_skip, pos_skip, W, b):
    b2d = b.reshape(1, DF)
    outs = []
    for c in range(NCHUNKS):
        sl = slice(c * CQ, (c + 1) * CQ)
        idx3, wnb = _topk(pos_skip[sl], pos)
        idx_flat = idx3.reshape(CQ * KNN)
        wnb_flat = wnb.reshape(CQ * KNN, 16)
        xi = _sc_interpolate(x, idx_flat, wnb_flat)
        outs.append(_mlp(xi, x_skip[sl], W, b2d))
    out = jnp.concatenate(outs, axis=0)
    return (out, pos_skip)
